# baseline (speedup 1.0000x reference)
import jax
import jax.numpy as jnp
from jax import lax
from jax.experimental import pallas as pl
from jax.experimental.pallas import tpu as pltpu
from jax.experimental.pallas import tpu_sc as plsc

B = 2; N = 8192; D = 512; H = 8; HD = 64; KC = 64; NC = 64; NR = 32
NROW = B * H * NC
SC_CORES = 2; SC_SUBCORES = 16; NWORK = SC_CORES * SC_SUBCORES
ROWS_PER_W = NROW // NWORK
D2 = 2 * D
F32 = jnp.float32
I32 = jnp.int32


def _k1a_body(x_ref, pp_ref, nw_ref, dirs_ref, freqs_ref, ropew_ref,
              wq_ref, wk_ref, wv_ref, q_ref, k_ref, v_ref):
    xb = x_ref[0]
    ms = jnp.mean(xb * xb, axis=-1, keepdims=True)
    xn = xb * lax.rsqrt(ms + 1e-6) * nw_ref[0][None, :]
    pp = pp_ref[0]
    pn = jnp.sqrt(jnp.sum(pp * pp, axis=-1, keepdims=True))
    upp = pp / jnp.maximum(pn, 1e-12)
    proj = lax.dot_general(upp, dirs_ref[...], (((1,), (1,)), ((), ())),
                           preferred_element_type=F32)
    ang = proj * freqs_ref[0][None, :]
    remb = jnp.concatenate([jnp.sin(ang), jnp.cos(ang)], axis=-1)
    rope = lax.dot_general(remb, ropew_ref[...], (((1,), (1,)), ((), ())),
                           preferred_element_type=F32)
    qk = xn + rope
    dn = (((1,), (0,)), ((), ()))
    q_ref[0] = lax.dot_general(qk, wq_ref[...], dn, preferred_element_type=F32)
    k_ref[0] = lax.dot_general(qk, wk_ref[...], dn, preferred_element_type=F32)
    v_ref[0] = lax.dot_general(xn, wv_ref[...], dn, preferred_element_type=F32)


def _k1a(x2, pp, norm_w, rope_dirs, rope_freqs, rope_W,
         wq_pad, wk_pad, wv_pad):
    nb = 1024
    grid = (B, N // nb)
    row = pl.BlockSpec((1, nb, D), lambda b, j: (b, j, 0))
    rowp = pl.BlockSpec((1, nb, D2), lambda b, j: (b, j, 0))
    w = pl.BlockSpec((D, D2), lambda b, j: (0, 0))
    return pl.pallas_call(
        _k1a_body,
        grid=grid,
        in_specs=[row,
                  pl.BlockSpec((1, nb, 3), lambda b, j: (b, j, 0)),
                  pl.BlockSpec((1, D), lambda b, j: (0, 0)),
                  pl.BlockSpec((NR, 3), lambda b, j: (0, 0)),
                  pl.BlockSpec((1, NR), lambda b, j: (0, 0)),
                  pl.BlockSpec((D, 2 * NR), lambda b, j: (0, 0)),
                  w, w, w],
        out_specs=[rowp, rowp, rowp],
        out_shape=[jax.ShapeDtypeStruct((B, N, D2), F32)] * 3,
    )(x2, pp, norm_w.reshape(1, D), rope_dirs, rope_freqs.reshape(1, NR),
      rope_W, wq_pad, wk_pad, wv_pad)


def _k1b_body(qf_ref, kf_ref, pp_ref, icf_ref, icp_ref, ocf_ref, ocp_ref,
              is_ref, os_ref):
    pp = pp_ref[0]
    pn = jnp.sqrt(jnp.sum(pp * pp, axis=-1, keepdims=True))
    upp = pp / jnp.maximum(pn, 1e-12)
    possq = jnp.sum(upp * upp, axis=-1, keepdims=True)

    def scores(featfull, cf_ref, cp_ref):
        feat = featfull[:, :HD]
        nrm = jnp.sqrt(jnp.sum(feat * feat, axis=-1, keepdims=True) + possq)
        rn = jnp.where(possq == 0.0, 0.0, 1.0 / jnp.maximum(nrm, 1e-12))
        fs = feat * rn
        us = upp * rn
        return (lax.dot_general(cf_ref[0], fs, (((1,), (1,)), ((), ())),
                                preferred_element_type=F32)
                + lax.dot_general(cp_ref[0], us, (((1,), (1,)), ((), ())),
                                  preferred_element_type=F32))

    is_ref[0, 0] = scores(qf_ref[...], icf_ref, icp_ref)
    os_ref[0, 0] = scores(kf_ref[...], ocf_ref, ocp_ref)


def _k1b(qf, kf, pp, icf, icp, ocf, ocp):
    nb = 2048
    jn = N // nb
    grid = (B, H, jn)
    tab = pl.BlockSpec((nb, 2 * HD),
                       lambda b, h, j: (b * (H * jn) + h * jn + j, 0))
    cf = pl.BlockSpec((1, NC, HD), lambda b, h, j: (h, 0, 0))
    cp = pl.BlockSpec((1, NC, 3), lambda b, h, j: (h, 0, 0))
    sct = pl.BlockSpec((1, 1, NC, nb), lambda b, h, j: (b, h, 0, j))
    return pl.pallas_call(
        _k1b_body,
        grid=grid,
        in_specs=[tab, tab,
                  pl.BlockSpec((1, nb, 3), lambda b, h, j: (b, j, 0)),
                  cf, cp, cf, cp],
        out_specs=[sct, sct],
        out_shape=[
            jax.ShapeDtypeStruct((B, H, NC, N), F32),
            jax.ShapeDtypeStruct((B, H, NC, N), F32),
        ],
    )(qf, kf, pp, icf, icp, ocf, ocp)


def _f32_keys(s):
    b = lax.bitcast_convert_type(s, I32)
    return jnp.where(b < 0, b ^ jnp.int32(0x7FFFFFFF), b)


def _bisect_t(keys):
    t0 = jnp.full((keys.shape[0], 1), -(2 ** 30), I32)

    def body(i, t):
        bit = lax.shift_left(jnp.int32(1), jnp.int32(30) - i)
        tp = t + bit
        cnt = jnp.sum((keys >= tp).astype(I32), axis=1, keepdims=True)
        return jnp.where(cnt >= KC, tp, t)

    return lax.fori_loop(0, 31, body, t0)


def _prefix_sum(mi):
    rr = mi.shape[0]
    z = jnp.zeros((rr, N), I32)
    k = 1
    while k < N:
        sh = jnp.concatenate([z[:, :k], mi[:, :N - k]], axis=1)
        mi = mi + sh
        k *= 2
    return mi


K2R = 64


def _k2_body(sin_ref, sout_ref, pin_ref, pout_ref, cnt_ref, cacc_ref):
    lanemod = lax.broadcasted_iota(I32, (K2R, N), 1) % 16

    def positions(keys):
        t = _bisect_t(keys)
        mask = keys >= t
        p = _prefix_sum(mask.astype(I32))
        pos = jnp.where(jnp.logical_and(mask, p <= KC), p - 1, KC + lanemod)
        return mask, pos

    mask_in, pos_in = positions(_f32_keys(sin_ref[...]))
    pin_ref[...] = pos_in
    i = pl.program_id(0)

    @pl.when(i % (NC // K2R) == 0)
    def _():
        cacc_ref[...] = jnp.zeros((1, N), F32)

    cacc_ref[...] += jnp.sum(mask_in.astype(F32), axis=0, keepdims=True)

    @pl.when(i % (NC // K2R) == NC // K2R - 1)
    def _():
        cnt_ref[0] = jnp.swapaxes(cacc_ref[...], 0, 1)
    _, pos_out = positions(_f32_keys(sout_ref[...]))
    pout_ref[...] = pos_out


def _k2(is_flat, os_flat):
    grid = (NROW // K2R,)
    blk = pl.BlockSpec((K2R, N), lambda i: (i, 0))
    return pl.pallas_call(
        _k2_body,
        grid=grid,
        in_specs=[blk, blk],
        out_specs=[blk, blk,
                   pl.BlockSpec((1, N, 1), lambda i: (i // (NC // K2R), 0, 0))],
        out_shape=[
            jax.ShapeDtypeStruct((NROW, N), I32),
            jax.ShapeDtypeStruct((NROW, N), I32),
            jax.ShapeDtypeStruct((B * H, N, 1), F32),
        ],
        scratch_shapes=[pltpu.VMEM((1, N), F32)],
    )(is_flat, os_flat)


IB = 128


def _extract_indices(prow_ref, idx_ref):
    iota16 = lax.iota(I32, 16)

    def body(j, carry):
        pos16 = prow_ref[pl.ds(j * 16, 16)]
        plsc.store_scatter(idx_ref, [pos16], iota16 + j * 16)
        return carry

    lax.fori_loop(0, N // 16, body, jnp.int32(0))


def _k3_body(pin_hbm, pout_hbm,
             qtab_hbm, ktab_hbm, vtab_hbm, smt_hbm,
             qcl_hbm, kcl_hbm, vcl_hbm, iidx_hbm,
             prow_in, prow_out, idx_in, idx_out, gidx_q, gidx_kv, gidx_sm,
             rows_q, rows_k, rows_v, gm_v,
             sem_pi, sem_po, sem_q, sem_k, sem_v, sem_g, sem_w):
    c = lax.axis_index("c")
    s = lax.axis_index("s")
    wid = c * SC_SUBCORES + s
    base = wid * ROWS_PER_W
    sm_off = c * N

    def row_body(rr, carry):
        r = base + rr
        bh_off = (r // NC) * N

        ld_in = pltpu.async_copy(pin_hbm.at[r], prow_in, sem_pi)
        ld_out = pltpu.async_copy(pout_hbm.at[r], prow_out, sem_po)

        ld_in.wait()
        _extract_indices(prow_in, idx_in)
        w_idx = pltpu.async_copy(idx_in, iidx_hbm.at[r], sem_w)
        for jj in range(KC // 16):
            gidx_q[pl.ds(jj * 16, 16)] = idx_in[pl.ds(jj * 16, 16)] + bh_off
        cq = pltpu.async_copy(qtab_hbm.at[gidx_q], rows_q, sem_q)

        ld_out.wait()
        _extract_indices(prow_out, idx_out)
        for jj in range(KC // 16):
            loc = idx_out[pl.ds(jj * 16, 16)]
            gidx_kv[pl.ds(jj * 16, 16)] = loc + bh_off
            gidx_sm[pl.ds(jj * 16, 16)] = loc + sm_off
        ck = pltpu.async_copy(ktab_hbm.at[gidx_kv], rows_k, sem_k)
        cv = pltpu.async_copy(vtab_hbm.at[gidx_kv], rows_v, sem_v)
        cg = pltpu.async_copy(smt_hbm.at[gidx_sm], gm_v, sem_g)

        cq.wait()
        w_q = pltpu.async_copy(rows_q, qcl_hbm.at[r], sem_w)
        ck.wait()
        w_k = pltpu.async_copy(rows_k, kcl_hbm.at[r], sem_w)
        cv.wait()
        cg.wait()
        for i in range(KC):
            rows_v[i, pl.ds(HD, 16)] = gm_v[i, pl.ds(0, 16)]
        w_v = pltpu.async_copy(rows_v, vcl_hbm.at[r], sem_w)
        w_idx.wait(); w_q.wait(); w_k.wait(); w_v.wait()
        return carry

    lax.fori_loop(0, ROWS_PER_W, row_body, jnp.int32(0))


def _k3(pos_in, pos_out, q_flat, k_flat, v_flat, smt):
    mesh = plsc.VectorSubcoreMesh(core_axis_name="c", subcore_axis_name="s")
    kern = pl.kernel(
        _k3_body,
        out_type=[
            jax.ShapeDtypeStruct((NROW, KC, 2 * HD), F32),
            jax.ShapeDtypeStruct((NROW, KC, 2 * HD), F32),
            jax.ShapeDtypeStruct((NROW, KC, 2 * HD), F32),
            jax.ShapeDtypeStruct((NROW, IB), I32),
        ],
        mesh=mesh,
        compiler_params=pltpu.CompilerParams(needs_layout_passes=False),
        scratch_types=[
            pltpu.VMEM((N,), I32),
            pltpu.VMEM((N,), I32),
            pltpu.VMEM((IB,), I32),
            pltpu.VMEM((IB,), I32),
            pltpu.VMEM((KC,), I32),
            pltpu.VMEM((KC,), I32),
            pltpu.VMEM((KC,), I32),
            pltpu.VMEM((KC, 2 * HD), F32),
            pltpu.VMEM((KC, 2 * HD), F32),
            pltpu.VMEM((KC, 2 * HD), F32),
            pltpu.VMEM((KC, 2 * HD), F32),
        ] + [pltpu.SemaphoreType.DMA] * 7,
    )
    return kern(pos_in, pos_out, q_flat, k_flat, v_flat, smt)


def _k4_body(q_ref, k_ref, v_ref, o_ref):
    cb = q_ref.shape[0]
    ones_col = jnp.ones((KC, 1), F32)
    for ci in range(cb):
        qc = q_ref[ci][:, :HD]
        kc = k_ref[ci][:, :HD]
        vc = v_ref[ci][:, :HD]
        nsc = (v_ref[ci][:, HD:HD + 1] == 0.0).astype(F32)
        logits = lax.dot_general(qc, kc, (((1,), (1,)), ((), ())),
                                 preferred_element_type=F32) * (KC ** -0.5)
        logits = logits - 1e9 * lax.dot_general(
            ones_col, nsc, (((1,), (1,)), ((), ())),
            preferred_element_type=F32)
        m = jnp.max(logits, axis=-1, keepdims=True)
        p = jnp.exp(logits - m)
        attn = p / jnp.sum(p, axis=-1, keepdims=True)
        attn = jnp.where(jnp.sum(nsc) == KC, 0.0, attn)
        oc = lax.dot_general(attn, vc, (((1,), (0,)), ((), ())),
                             preferred_element_type=F32)
        o_ref[ci] = jnp.concatenate([oc, jnp.zeros((KC, HD), F32)], axis=-1)


def _k4(q_cl, k_cl, v_cl):
    CB = 128
    grid = (NROW // CB,)
    blkp = pl.BlockSpec((CB, KC, 2 * HD), lambda i: (i, 0, 0))
    return pl.pallas_call(
        _k4_body,
        grid=grid,
        in_specs=[blkp, blkp, blkp],
        out_specs=blkp,
        out_shape=jax.ShapeDtypeStruct((NROW, KC, 2 * HD), F32),
    )(q_cl, k_cl, v_cl)


PAIRS = B * H
ROWS_PER_PAIR = NC * KC
CHUNK = 128
CH_PER_TILE = ROWS_PER_PAIR // SC_SUBCORES // CHUNK


def _k5_body(o_hbm, idx_hbm, z_hbm, acc_hbm, rows_v, idx_v, acc_s):
    c = lax.axis_index("c")
    s = lax.axis_index("s")
    zslice = pl.ds(s * (N // SC_SUBCORES), N // SC_SUBCORES)
    for p in range(PAIRS // SC_CORES):
        bh = p * SC_CORES + c
        pltpu.sync_copy(z_hbm.at[zslice], acc_s.at[zslice])
        plsc.subcore_barrier()
        for j in range(CH_PER_TILE):
            ch = bh * (ROWS_PER_PAIR // CHUNK) + s * CH_PER_TILE + j
            pltpu.sync_copy(o_hbm.at[ch], rows_v.at[j])
            pltpu.sync_copy(idx_hbm.at[ch], idx_v.at[j])
        for j in range(CH_PER_TILE):
            pltpu.sync_copy(rows_v.at[j], acc_s.at[idx_v.at[j]], add=True)
        plsc.subcore_barrier()
        pltpu.sync_copy(acc_s.at[zslice], acc_hbm.at[bh].at[zslice])
        plsc.subcore_barrier()


def _k5(o3, idx3, zeros_hbm):
    mesh = plsc.VectorSubcoreMesh(core_axis_name="c", subcore_axis_name="s")
    kern = pl.kernel(
        _k5_body,
        out_type=[jax.ShapeDtypeStruct((PAIRS, N, 2 * HD), F32)],
        mesh=mesh,
        scratch_types=[
            pltpu.VMEM((CH_PER_TILE, CHUNK, 2 * HD), F32),
            pltpu.VMEM((CH_PER_TILE, CHUNK), I32),
            pltpu.MemorySpace.VMEM_SHARED((N, 2 * HD), F32),
        ],
    )
    return kern(o3, idx3, zeros_hbm)[0]


def _k6_body(x_ref, acc_ref, cnt_ref, y_ref):
    a = acc_ref[0].reshape(N // H, H, 2 * HD)
    cdiv = jnp.maximum(cnt_ref[0], 1.0).reshape(N // H, H, 1)
    for j2 in range(H):
        aj = a[:, j2, :HD]
        cj = cdiv[:, j2, :]
        y_ref[0, :, j2 * HD:(j2 + 1) * HD] = (
            x_ref[0][:, j2 * HD:(j2 + 1) * HD] + aj / cj)


def _k6(x2, acc, cnt_col):
    grid = (B, H)
    return pl.pallas_call(
        _k6_body,
        grid=grid,
        in_specs=[
            pl.BlockSpec((1, N // H, D), lambda b, h: (b, h, 0)),
            pl.BlockSpec((1, N, 2 * HD), lambda b, h: (b * H + h, 0, 0)),
            pl.BlockSpec((1, N, 1), lambda b, h: (b * H + h, 0, 0)),
        ],
        out_specs=pl.BlockSpec((1, N // H, D), lambda b, h: (b, h, 0)),
        out_shape=jax.ShapeDtypeStruct((B, N, D), F32),
    )(x2, acc, cnt_col)


def kernel(x, point_positions, neuron_pad_mask, spike_mask, norm_w, rope_W,
           rope_dirs, rope_freqs, Wq, Wk, Wv, input_centroids,
           output_centroids):
    Bx, Tx, Nx, Dx = x.shape
    x2 = x.reshape(B, N, D)
    pp = point_positions
    sm32 = spike_mask.reshape(B, N).astype(F32)

    def norm_split(cen):
        cn = cen / jnp.maximum(
            jnp.linalg.norm(cen, axis=-1, keepdims=True), 1e-12)
        return cn[..., :HD], cn[..., HD:]

    icf, icp = norm_split(input_centroids)
    ocf, ocp = norm_split(output_centroids)

    def pad_w(w):
        wt = w.T.reshape(D, H, HD)
        return jnp.concatenate(
            [wt, jnp.zeros((D, H, HD), F32)], axis=-1).reshape(D, D2)

    q_full, k_full, v_full = _k1a(x2, pp, norm_w, rope_dirs, rope_freqs,
                                  rope_W, pad_w(Wq), pad_w(Wk), pad_w(Wv))
    qf = q_full.reshape(B * H * N, 2 * HD)
    kf = k_full.reshape(B * H * N, 2 * HD)
    vf = v_full.reshape(B * H * N, 2 * HD)
    isT, osT = _k1b(qf, kf, pp, icf, icp, ocf, ocp)
    pos_in, pos_out, cnt = _k2(isT.reshape(NROW, N), osT.reshape(NROW, N))
    smt = jnp.broadcast_to(sm32.reshape(B * N, 1), (B * N, 2 * HD))
    q_cl, k_cl, v_cl, iidx = _k3(pos_in, pos_out, qf, kf, vf, smt)
    in_idx = iidx[:, :KC]
    o_cl = _k4(q_cl, k_cl, v_cl)
    zeros_hbm = jnp.zeros((N, 2 * HD), F32)
    acc = _k5(o_cl.reshape(NROW * KC // CHUNK, CHUNK, 2 * HD),
              in_idx.reshape(NROW * KC // CHUNK, CHUNK), zeros_hbm)
    y = _k6(x2, acc, cnt)
    return y.reshape(Bx, Tx, Nx, Dx)


# speedup vs baseline: 1.0007x; 1.0007x over previous
import jax
import jax.numpy as jnp
from jax import lax
from jax.experimental import pallas as pl
from jax.experimental.pallas import tpu as pltpu
from jax.experimental.pallas import tpu_sc as plsc

B = 2; N = 8192; D = 512; H = 8; HD = 64; KC = 64; NC = 64; NR = 32
NROW = B * H * NC
SC_CORES = 2; SC_SUBCORES = 16; NWORK = SC_CORES * SC_SUBCORES
ROWS_PER_W = NROW // NWORK
D2 = 2 * D
F32 = jnp.float32
I32 = jnp.int32


def _k1a_body(x_ref, pp_ref, nw_ref, dirs_ref, freqs_ref, ropew_ref,
              wq_ref, wk_ref, wv_ref, q_ref, k_ref, v_ref):
    xb = x_ref[0]
    ms = jnp.mean(xb * xb, axis=-1, keepdims=True)
    xn = xb * lax.rsqrt(ms + 1e-6) * nw_ref[0][None, :]
    pp = pp_ref[0]
    pn = jnp.sqrt(jnp.sum(pp * pp, axis=-1, keepdims=True))
    upp = pp / jnp.maximum(pn, 1e-12)
    proj = lax.dot_general(upp, dirs_ref[...], (((1,), (1,)), ((), ())),
                           preferred_element_type=F32)
    ang = proj * freqs_ref[0][None, :]
    remb = jnp.concatenate([jnp.sin(ang), jnp.cos(ang)], axis=-1)
    rope = lax.dot_general(remb, ropew_ref[...], (((1,), (1,)), ((), ())),
                           preferred_element_type=F32)
    qk = xn + rope
    dn = (((1,), (0,)), ((), ()))
    q_ref[0] = lax.dot_general(qk, wq_ref[...], dn, preferred_element_type=F32)
    k_ref[0] = lax.dot_general(qk, wk_ref[...], dn, preferred_element_type=F32)
    v_ref[0] = lax.dot_general(xn, wv_ref[...], dn, preferred_element_type=F32)


def _k1a(x2, pp, norm_w, rope_dirs, rope_freqs, rope_W,
         wq_pad, wk_pad, wv_pad):
    nb = 1024
    grid = (B, N // nb)
    row = pl.BlockSpec((1, nb, D), lambda b, j: (b, j, 0))
    rowp = pl.BlockSpec((1, nb, D2), lambda b, j: (b, j, 0))
    w = pl.BlockSpec((D, D2), lambda b, j: (0, 0))
    return pl.pallas_call(
        _k1a_body,
        grid=grid,
        in_specs=[row,
                  pl.BlockSpec((1, nb, 3), lambda b, j: (b, j, 0)),
                  pl.BlockSpec((1, D), lambda b, j: (0, 0)),
                  pl.BlockSpec((NR, 3), lambda b, j: (0, 0)),
                  pl.BlockSpec((1, NR), lambda b, j: (0, 0)),
                  pl.BlockSpec((D, 2 * NR), lambda b, j: (0, 0)),
                  w, w, w],
        out_specs=[rowp, rowp, rowp],
        out_shape=[jax.ShapeDtypeStruct((B, N, D2), F32)] * 3,
    )(x2, pp, norm_w.reshape(1, D), rope_dirs, rope_freqs.reshape(1, NR),
      rope_W, wq_pad, wk_pad, wv_pad)


def _k1b_body(qf_ref, kf_ref, pp_ref, icf_ref, icp_ref, ocf_ref, ocp_ref,
              is_ref, os_ref):
    pp = pp_ref[0]
    pn = jnp.sqrt(jnp.sum(pp * pp, axis=-1, keepdims=True))
    upp = pp / jnp.maximum(pn, 1e-12)
    possq = jnp.sum(upp * upp, axis=-1, keepdims=True)

    def scores(featfull, cf_ref, cp_ref):
        feat = featfull[:, :HD]
        nrm = jnp.sqrt(jnp.sum(feat * feat, axis=-1, keepdims=True) + possq)
        rn = jnp.where(possq == 0.0, 0.0, 1.0 / jnp.maximum(nrm, 1e-12))
        fs = feat * rn
        us = upp * rn
        return (lax.dot_general(cf_ref[0], fs, (((1,), (1,)), ((), ())),
                                preferred_element_type=F32)
                + lax.dot_general(cp_ref[0], us, (((1,), (1,)), ((), ())),
                                  preferred_element_type=F32))

    is_ref[0, 0] = scores(qf_ref[...], icf_ref, icp_ref)
    os_ref[0, 0] = scores(kf_ref[...], ocf_ref, ocp_ref)


def _k1b(qf, kf, pp, icf, icp, ocf, ocp):
    nb = 2048
    jn = N // nb
    grid = (B, H, jn)
    tab = pl.BlockSpec((nb, 2 * HD),
                       lambda b, h, j: (b * (H * jn) + h * jn + j, 0))
    cf = pl.BlockSpec((1, NC, HD), lambda b, h, j: (h, 0, 0))
    cp = pl.BlockSpec((1, NC, 3), lambda b, h, j: (h, 0, 0))
    sct = pl.BlockSpec((1, 1, NC, nb), lambda b, h, j: (b, h, 0, j))
    return pl.pallas_call(
        _k1b_body,
        grid=grid,
        in_specs=[tab, tab,
                  pl.BlockSpec((1, nb, 3), lambda b, h, j: (b, j, 0)),
                  cf, cp, cf, cp],
        out_specs=[sct, sct],
        out_shape=[
            jax.ShapeDtypeStruct((B, H, NC, N), F32),
            jax.ShapeDtypeStruct((B, H, NC, N), F32),
        ],
    )(qf, kf, pp, icf, icp, ocf, ocp)


def _f32_keys(s):
    b = lax.bitcast_convert_type(s, I32)
    return jnp.where(b < 0, b ^ jnp.int32(0x7FFFFFFF), b)


def _bisect_t(keys):
    t0 = jnp.full((keys.shape[0], 1), -(2 ** 30), I32)

    def body(i, t):
        bit = lax.shift_left(jnp.int32(1), jnp.int32(30) - i)
        tp = t + bit
        cnt = jnp.sum((keys >= tp).astype(I32), axis=1, keepdims=True)
        return jnp.where(cnt >= KC, tp, t)

    return lax.fori_loop(0, 31, body, t0)


def _prefix_sum(mi):
    rr = mi.shape[0]
    z = jnp.zeros((rr, N), I32)
    k = 1
    while k < N:
        sh = jnp.concatenate([z[:, :k], mi[:, :N - k]], axis=1)
        mi = mi + sh
        k *= 2
    return mi


K2R = 64


def _k2_body(sin_ref, sout_ref, pin_ref, pout_ref, cnt_ref, cacc_ref):
    lanemod = lax.broadcasted_iota(I32, (K2R, N), 1) % 16

    def positions(keys):
        t = _bisect_t(keys)
        mask = keys >= t
        p = _prefix_sum(mask.astype(I32))
        pos = jnp.where(jnp.logical_and(mask, p <= KC), p - 1, KC + lanemod)
        return mask, pos

    mask_in, pos_in = positions(_f32_keys(sin_ref[...]))
    pin_ref[...] = pos_in
    i = pl.program_id(0)

    @pl.when(i % (NC // K2R) == 0)
    def _():
        cacc_ref[...] = jnp.zeros((1, N), F32)

    cacc_ref[...] += jnp.sum(mask_in.astype(F32), axis=0, keepdims=True)

    @pl.when(i % (NC // K2R) == NC // K2R - 1)
    def _():
        cnt_ref[0] = jnp.swapaxes(cacc_ref[...], 0, 1)
    _, pos_out = positions(_f32_keys(sout_ref[...]))
    pout_ref[...] = pos_out


def _k2(is_flat, os_flat):
    grid = (NROW // K2R,)
    blk = pl.BlockSpec((K2R, N), lambda i: (i, 0))
    return pl.pallas_call(
        _k2_body,
        grid=grid,
        in_specs=[blk, blk],
        out_specs=[blk, blk,
                   pl.BlockSpec((1, N, 1), lambda i: (i // (NC // K2R), 0, 0))],
        out_shape=[
            jax.ShapeDtypeStruct((NROW, N), I32),
            jax.ShapeDtypeStruct((NROW, N), I32),
            jax.ShapeDtypeStruct((B * H, N, 1), F32),
        ],
        scratch_shapes=[pltpu.VMEM((1, N), F32)],
    )(is_flat, os_flat)


IB = 128


def _extract_indices(prow_ref, idx_ref):
    iota16 = lax.iota(I32, 16)

    def body(j, carry):
        pos16 = prow_ref[pl.ds(j * 16, 16)]
        plsc.store_scatter(idx_ref, [pos16], iota16 + j * 16)
        return carry

    lax.fori_loop(0, N // 16, body, jnp.int32(0))


def _k3_body(pin_hbm, pout_hbm,
             qtab_hbm, ktab_hbm, vtab_hbm, smt_hbm,
             qcl_hbm, kcl_hbm, vcl_hbm, iidx_hbm,
             prow_in, prow_out, idx_in, idx_out, gidx_q, gidx_kv, gidx_sm,
             rows_q, rows_k, rows_v, gm_v,
             sem_pi, sem_po, sem_q, sem_k, sem_v, sem_g, sem_w):
    c = lax.axis_index("c")
    s = lax.axis_index("s")
    wid = c * SC_SUBCORES + s
    base = wid * ROWS_PER_W
    sm_off = c * N

    def row_body(rr, carry):
        r = base + rr
        bh_off = (r // NC) * N

        ld_in = pltpu.async_copy(pin_hbm.at[r], prow_in, sem_pi)
        ld_out = pltpu.async_copy(pout_hbm.at[r], prow_out, sem_po)

        ld_in.wait()
        _extract_indices(prow_in, idx_in)
        w_idx = pltpu.async_copy(idx_in, iidx_hbm.at[r], sem_w)
        for jj in range(KC // 16):
            gidx_q[pl.ds(jj * 16, 16)] = idx_in[pl.ds(jj * 16, 16)] + bh_off
        cq = pltpu.async_copy(qtab_hbm.at[gidx_q], rows_q, sem_q)

        ld_out.wait()
        _extract_indices(prow_out, idx_out)
        for jj in range(KC // 16):
            loc = idx_out[pl.ds(jj * 16, 16)]
            gidx_kv[pl.ds(jj * 16, 16)] = loc + bh_off
            gidx_sm[pl.ds(jj * 16, 16)] = loc + sm_off
        ck = pltpu.async_copy(ktab_hbm.at[gidx_kv], rows_k, sem_k)
        cv = pltpu.async_copy(vtab_hbm.at[gidx_kv], rows_v, sem_v)
        cg = pltpu.async_copy(smt_hbm.at[gidx_sm], gm_v, sem_g)

        cq.wait()
        w_q = pltpu.async_copy(rows_q, qcl_hbm.at[r], sem_w)
        ck.wait()
        w_k = pltpu.async_copy(rows_k, kcl_hbm.at[r], sem_w)
        cv.wait()
        cg.wait()
        for i in range(KC):
            rows_v[i, pl.ds(HD, 16)] = gm_v[i, pl.ds(0, 16)]
        w_v = pltpu.async_copy(rows_v, vcl_hbm.at[r], sem_w)
        w_idx.wait(); w_q.wait(); w_k.wait(); w_v.wait()
        return carry

    lax.fori_loop(0, ROWS_PER_W, row_body, jnp.int32(0))


def _k3(pos_in, pos_out, q_flat, k_flat, v_flat, smt):
    mesh = plsc.VectorSubcoreMesh(core_axis_name="c", subcore_axis_name="s")
    kern = pl.kernel(
        _k3_body,
        out_type=[
            jax.ShapeDtypeStruct((NROW, KC, 2 * HD), F32),
            jax.ShapeDtypeStruct((NROW, KC, 2 * HD), F32),
            jax.ShapeDtypeStruct((NROW, KC, 2 * HD), F32),
            jax.ShapeDtypeStruct((NROW, IB), I32),
        ],
        mesh=mesh,
        compiler_params=pltpu.CompilerParams(needs_layout_passes=False),
        scratch_types=[
            pltpu.VMEM((N,), I32),
            pltpu.VMEM((N,), I32),
            pltpu.VMEM((IB,), I32),
            pltpu.VMEM((IB,), I32),
            pltpu.VMEM((KC,), I32),
            pltpu.VMEM((KC,), I32),
            pltpu.VMEM((KC,), I32),
            pltpu.VMEM((KC, 2 * HD), F32),
            pltpu.VMEM((KC, 2 * HD), F32),
            pltpu.VMEM((KC, 2 * HD), F32),
            pltpu.VMEM((KC, 2 * HD), F32),
        ] + [pltpu.SemaphoreType.DMA] * 7,
    )
    return kern(pos_in, pos_out, q_flat, k_flat, v_flat, smt)


def _k4_body(q_ref, k_ref, v_ref, o_ref):
    cb = q_ref.shape[0]
    ones_col = jnp.ones((KC, 1), F32)
    for ci in range(cb):
        qc = q_ref[ci][:, :HD]
        kc = k_ref[ci][:, :HD]
        vc = v_ref[ci][:, :HD]
        nsc = (v_ref[ci][:, HD:HD + 1] == 0.0).astype(F32)
        logits = lax.dot_general(qc, kc, (((1,), (1,)), ((), ())),
                                 preferred_element_type=F32) * (KC ** -0.5)
        logits = logits - 1e9 * lax.dot_general(
            ones_col, nsc, (((1,), (1,)), ((), ())),
            preferred_element_type=F32)
        m = jnp.max(logits, axis=-1, keepdims=True)
        p = jnp.exp(logits - m)
        attn = p / jnp.sum(p, axis=-1, keepdims=True)
        attn = jnp.where(jnp.sum(nsc) == KC, 0.0, attn)
        oc = lax.dot_general(attn, vc, (((1,), (0,)), ((), ())),
                             preferred_element_type=F32)
        o_ref[ci] = jnp.concatenate([oc, jnp.zeros((KC, HD), F32)], axis=-1)


def _k4(q_cl, k_cl, v_cl):
    CB = 64
    grid = (NROW // CB,)
    blkp = pl.BlockSpec((CB, KC, 2 * HD), lambda i: (i, 0, 0))
    return pl.pallas_call(
        _k4_body,
        grid=grid,
        in_specs=[blkp, blkp, blkp],
        out_specs=blkp,
        out_shape=jax.ShapeDtypeStruct((NROW, KC, 2 * HD), F32),
    )(q_cl, k_cl, v_cl)


PAIRS = B * H
ROWS_PER_PAIR = NC * KC
CHUNK = 128
CH_PER_TILE = ROWS_PER_PAIR // SC_SUBCORES // CHUNK


def _k5_body(o_hbm, idx_hbm, z_hbm, acc_hbm, rows_v, idx_v, acc_s):
    c = lax.axis_index("c")
    s = lax.axis_index("s")
    zslice = pl.ds(s * (N // SC_SUBCORES), N // SC_SUBCORES)
    for p in range(PAIRS // SC_CORES):
        bh = p * SC_CORES + c
        pltpu.sync_copy(z_hbm.at[zslice], acc_s.at[zslice])
        plsc.subcore_barrier()
        for j in range(CH_PER_TILE):
            ch = bh * (ROWS_PER_PAIR // CHUNK) + s * CH_PER_TILE + j
            pltpu.sync_copy(o_hbm.at[ch], rows_v.at[j])
            pltpu.sync_copy(idx_hbm.at[ch], idx_v.at[j])
        for j in range(CH_PER_TILE):
            pltpu.sync_copy(rows_v.at[j], acc_s.at[idx_v.at[j]], add=True)
        plsc.subcore_barrier()
        pltpu.sync_copy(acc_s.at[zslice], acc_hbm.at[bh].at[zslice])
        plsc.subcore_barrier()


def _k5(o3, idx3, zeros_hbm):
    mesh = plsc.VectorSubcoreMesh(core_axis_name="c", subcore_axis_name="s")
    kern = pl.kernel(
        _k5_body,
        out_type=[jax.ShapeDtypeStruct((PAIRS, N, 2 * HD), F32)],
        mesh=mesh,
        scratch_types=[
            pltpu.VMEM((CH_PER_TILE, CHUNK, 2 * HD), F32),
            pltpu.VMEM((CH_PER_TILE, CHUNK), I32),
            pltpu.MemorySpace.VMEM_SHARED((N, 2 * HD), F32),
        ],
    )
    return kern(o3, idx3, zeros_hbm)[0]


def _k6_body(x_ref, acc_ref, cnt_ref, y_ref):
    a = acc_ref[0].reshape(N // H, H, 2 * HD)
    cdiv = jnp.maximum(cnt_ref[0], 1.0).reshape(N // H, H, 1)
    for j2 in range(H):
        aj = a[:, j2, :HD]
        cj = cdiv[:, j2, :]
        y_ref[0, :, j2 * HD:(j2 + 1) * HD] = (
            x_ref[0][:, j2 * HD:(j2 + 1) * HD] + aj / cj)


def _k6(x2, acc, cnt_col):
    grid = (B, H)
    return pl.pallas_call(
        _k6_body,
        grid=grid,
        in_specs=[
            pl.BlockSpec((1, N // H, D), lambda b, h: (b, h, 0)),
            pl.BlockSpec((1, N, 2 * HD), lambda b, h: (b * H + h, 0, 0)),
            pl.BlockSpec((1, N, 1), lambda b, h: (b * H + h, 0, 0)),
        ],
        out_specs=pl.BlockSpec((1, N // H, D), lambda b, h: (b, h, 0)),
        out_shape=jax.ShapeDtypeStruct((B, N, D), F32),
    )(x2, acc, cnt_col)


def kernel(x, point_positions, neuron_pad_mask, spike_mask, norm_w, rope_W,
           rope_dirs, rope_freqs, Wq, Wk, Wv, input_centroids,
           output_centroids):
    Bx, Tx, Nx, Dx = x.shape
    x2 = x.reshape(B, N, D)
    pp = point_positions
    sm32 = spike_mask.reshape(B, N).astype(F32)

    def norm_split(cen):
        cn = cen / jnp.maximum(
            jnp.linalg.norm(cen, axis=-1, keepdims=True), 1e-12)
        return cn[..., :HD], cn[..., HD:]

    icf, icp = norm_split(input_centroids)
    ocf, ocp = norm_split(output_centroids)

    def pad_w(w):
        wt = w.T.reshape(D, H, HD)
        return jnp.concatenate(
            [wt, jnp.zeros((D, H, HD), F32)], axis=-1).reshape(D, D2)

    q_full, k_full, v_full = _k1a(x2, pp, norm_w, rope_dirs, rope_freqs,
                                  rope_W, pad_w(Wq), pad_w(Wk), pad_w(Wv))
    qf = q_full.reshape(B * H * N, 2 * HD)
    kf = k_full.reshape(B * H * N, 2 * HD)
    vf = v_full.reshape(B * H * N, 2 * HD)
    isT, osT = _k1b(qf, kf, pp, icf, icp, ocf, ocp)
    pos_in, pos_out, cnt = _k2(isT.reshape(NROW, N), osT.reshape(NROW, N))
    smt = jnp.broadcast_to(sm32.reshape(B * N, 1), (B * N, 2 * HD))
    q_cl, k_cl, v_cl, iidx = _k3(pos_in, pos_out, qf, kf, vf, smt)
    in_idx = iidx[:, :KC]
    o_cl = _k4(q_cl, k_cl, v_cl)
    zeros_hbm = jnp.zeros((N, 2 * HD), F32)
    acc = _k5(o_cl.reshape(NROW * KC // CHUNK, CHUNK, 2 * HD),
              in_idx.reshape(NROW * KC // CHUNK, CHUNK), zeros_hbm)
    y = _k6(x2, acc, cnt)
    return y.reshape(Bx, Tx, Nx, Dx)
